# Initial kernel scaffold; baseline (speedup 1.0000x reference)
import jax
import jax.numpy as jnp
from jax.experimental import pallas as pl


def kernel(input_seq, input_len, emb_table, W_ih_f, W_hh_f, b_ih_f, b_hh_f, W_ih_b, W_hh_b, b_ih_b, b_hh_b):
    raise NotImplementedError("write your pallas kernel here")


# speedup vs baseline: 3.1442x; 3.1442x over previous
import functools

import jax
import jax.numpy as jnp
from jax import lax
from jax.experimental import pallas as pl
from jax.experimental.pallas import tpu as pltpu
from jax.experimental.pallas import tpu_sc as plsc

L = 50
B = 1024
EMBED = 64
HIDDEN = 128


def _make_sc_gather(embed, n_ids):
    info = plsc.get_sparse_core_info()
    nc, ns = info.num_cores, info.num_subcores
    nw = nc * ns
    assert n_ids % nw == 0
    b_per_w = n_ids // nw
    assert b_per_w % 8 == 0

    mesh = plsc.VectorSubcoreMesh(core_axis_name="c", subcore_axis_name="s")

    @functools.partial(
        pl.kernel,
        mesh=mesh,
        out_type=jax.ShapeDtypeStruct((n_ids, embed), jnp.float32),
        scratch_types=[
            pltpu.VMEM((b_per_w,), jnp.int32),
            pltpu.VMEM((b_per_w, embed), jnp.float32),
            pltpu.SemaphoreType.DMA,
        ],
        compiler_params=pltpu.CompilerParams(use_tc_tiling_on_sc=False),
    )
    def gather(table_hbm, idx_hbm, out_hbm, idx_v, rows_v, sem):
        wid = lax.axis_index("s") * nc + lax.axis_index("c")
        base = wid * b_per_w
        pltpu.sync_copy(idx_hbm.at[pl.ds(base, b_per_w)], idx_v)
        pltpu.async_copy(table_hbm.at[idx_v], rows_v, sem).wait()
        pltpu.sync_copy(rows_v, out_hbm.at[pl.ds(base, b_per_w)])

    return gather


def _gru_tc_body(emb_ref, wih_f, whh_f, bih_f, bhh_f, wih_b, whh_b, bih_b,
                 bhh_b, out_ref):
    Bb = emb_ref.shape[1]
    H = HIDDEN

    wihf = wih_f[...]
    whhf = whh_f[...]
    wihb = wih_b[...]
    whhb = whh_b[...]
    bihf = bih_f[...]
    bhhf = bhh_f[...]
    bihb = bih_b[...]
    bhhb = bhh_b[...]

    def gru_step(x, h, wih, whh, bih, bhh):
        xp = jnp.dot(x, wih, preferred_element_type=jnp.float32) + bih
        gh = jnp.dot(h, whh, preferred_element_type=jnp.float32) + bhh
        r = jax.nn.sigmoid(xp[:, 0:H] + gh[:, 0:H])
        z = jax.nn.sigmoid(xp[:, H:2 * H] + gh[:, H:2 * H])
        n = jnp.tanh(xp[:, 2 * H:3 * H] + r * gh[:, 2 * H:3 * H])
        return (1.0 - z) * n + z * h

    def step(t, carry):
        h_f, h_b = carry
        h_f = gru_step(emb_ref[t], h_f, wihf, whhf, bihf, bhhf)
        out_ref[t, :, 0:H] = h_f
        tb = L - 1 - t
        h_b = gru_step(emb_ref[tb], h_b, wihb, whhb, bihb, bhhb)
        out_ref[tb, :, H:2 * H] = h_b
        return h_f, h_b

    h0 = jnp.zeros((Bb, H), jnp.float32)
    lax.fori_loop(0, L, step, (h0, h0), unroll=False)


def _make_tc_gru(bb):
    grid = (B // bb,)
    full = lambda i: (0, 0)
    return pl.pallas_call(
        _gru_tc_body,
        grid=grid,
        in_specs=[
            pl.BlockSpec((L, bb, EMBED), lambda i: (0, i, 0)),
            pl.BlockSpec((EMBED, 3 * HIDDEN), full),
            pl.BlockSpec((HIDDEN, 3 * HIDDEN), full),
            pl.BlockSpec((1, 3 * HIDDEN), full),
            pl.BlockSpec((1, 3 * HIDDEN), full),
            pl.BlockSpec((EMBED, 3 * HIDDEN), full),
            pl.BlockSpec((HIDDEN, 3 * HIDDEN), full),
            pl.BlockSpec((1, 3 * HIDDEN), full),
            pl.BlockSpec((1, 3 * HIDDEN), full),
        ],
        out_specs=pl.BlockSpec((L, bb, 2 * HIDDEN), lambda i: (0, i, 0)),
        out_shape=jax.ShapeDtypeStruct((L, B, 2 * HIDDEN), jnp.float32),
        compiler_params=pltpu.CompilerParams(
            dimension_semantics=("arbitrary",),
        ),
    )


_BB = 256


def kernel(input_seq, input_len, emb_table, W_ih_f, W_hh_f, b_ih_f, b_hh_f,
           W_ih_b, W_hh_b, b_ih_b, b_hh_b):
    del input_len
    embed = emb_table.shape[1]
    ids = input_seq.reshape(-1).astype(jnp.int32)

    rows = _make_sc_gather(embed, ids.shape[0])(emb_table, ids)
    emb = rows.reshape(L, B, embed)

    out = _make_tc_gru(_BB)(
        emb,
        W_ih_f.T, W_hh_f.T, b_ih_f[None, :], b_hh_f[None, :],
        W_ih_b.T, W_hh_b.T, b_ih_b[None, :], b_hh_b[None, :])
    return out
